# baseline (speedup 1.0000x reference)
import numpy as np
import jax
import jax.numpy as jnp
from jax.experimental import pallas as pl

N_USERS = 2048
N_ITEMS = 1024
NR = 160000
NSUP = 5
NCLS = 5
NBAS = 3
INPUT_DIM = 512
CH = 100
CHP = 128
SIDE_H = 64
ENC = 128

_HI = jax.lax.Precision.HIGHEST


def _mask_constants():
    mkey = jax.random.key(12345)
    k1, k2, k3 = jax.random.split(mkey, 3)
    perm = jax.random.permutation(k1, NR)
    num_mask = 48000
    num_noise = 4800
    num_token = 43200
    mask_nodes = perm[:num_mask]
    perm_mask = jax.random.permutation(k2, num_mask)
    token_nodes = mask_nodes[perm_mask[:num_token]]
    noise_nodes = mask_nodes[perm_mask[num_mask - num_noise:]]
    noise_chosen = jax.random.permutation(k3, NR)[:num_noise]
    return (np.asarray(token_nodes), np.asarray(noise_nodes),
            np.asarray(noise_chosen))


_TOKEN_NODES, _NOISE_NODES, _NOISE_CHOSEN = _mask_constants()


def _k1_body(ui_ref, ii_ref, wg_ref, tu_ref, tv_ref):
    w = wg_ref[0]
    tu_ref[0] = jnp.dot(ui_ref[...], w, preferred_element_type=jnp.float32,
                        precision=_HI)
    tv_ref[0] = jnp.dot(ii_ref[...], w, preferred_element_type=jnp.float32,
                        precision=_HI)


def _input_transforms(user_inputs, item_inputs, wg_pad):
    return pl.pallas_call(
        _k1_body,
        grid=(NSUP,),
        in_specs=[
            pl.BlockSpec((N_USERS, INPUT_DIM), lambda i: (0, 0)),
            pl.BlockSpec((N_ITEMS, INPUT_DIM), lambda i: (0, 0)),
            pl.BlockSpec((1, INPUT_DIM, CHP), lambda i: (i, 0, 0)),
        ],
        out_specs=[
            pl.BlockSpec((1, N_USERS, CHP), lambda i: (i, 0, 0)),
            pl.BlockSpec((1, N_ITEMS, CHP), lambda i: (i, 0, 0)),
        ],
        out_shape=[
            jax.ShapeDtypeStruct((NSUP, N_USERS, CHP), jnp.float32),
            jax.ShapeDtypeStruct((NSUP, N_ITEMS, CHP), jnp.float32),
        ],
    )(user_inputs, item_inputs, wg_pad)


def _spmm_body(sup_ref, t_ref, out_ref):
    out_ref[0] = jnp.dot(sup_ref[0], t_ref[0],
                         preferred_element_type=jnp.float32, precision=_HI)


def _spmm(sup, t, n_rows, n_cols, bu):
    return pl.pallas_call(
        _spmm_body,
        grid=(NSUP, n_rows // bu),
        in_specs=[
            pl.BlockSpec((1, bu, n_cols), lambda i, j: (i, j, 0)),
            pl.BlockSpec((1, n_cols, CHP), lambda i, j: (i, 0, 0)),
        ],
        out_specs=pl.BlockSpec((1, bu, CHP), lambda i, j: (i, j, 0)),
        out_shape=jax.ShapeDtypeStruct((NSUP, n_rows, CHP), jnp.float32),
    )(sup, t)


def _embed_body(uh_ref, ih_ref, w2u_ref, w2v_ref, us_ref, is_ref,
                w1u_ref, b1u_ref, w1v_ref, b1v_ref, w2us_ref, w2vs_ref,
                ue_ref, ie_ref):
    ue = jnp.zeros((N_USERS, ENC), jnp.float32)
    ie = jnp.zeros((N_ITEMS, ENC), jnp.float32)
    for i in range(NSUP):
        ue += jnp.dot(jnp.maximum(uh_ref[i], 0.0), w2u_ref[i],
                      preferred_element_type=jnp.float32, precision=_HI)
        ie += jnp.dot(jnp.maximum(ih_ref[i], 0.0), w2v_ref[i],
                      preferred_element_type=jnp.float32, precision=_HI)
    us = jnp.maximum(jnp.dot(us_ref[...], w1u_ref[...],
                             preferred_element_type=jnp.float32,
                             precision=_HI) + b1u_ref[...], 0.0)
    vs = jnp.maximum(jnp.dot(is_ref[...], w1v_ref[...],
                             preferred_element_type=jnp.float32,
                             precision=_HI) + b1v_ref[...], 0.0)
    ue += jnp.dot(us, w2us_ref[...], preferred_element_type=jnp.float32,
                  precision=_HI)
    ie += jnp.dot(vs, w2vs_ref[...], preferred_element_type=jnp.float32,
                  precision=_HI)
    ue_ref[...] = ue
    ie_ref[...] = ie


def _embeddings(uh, ih, w2u_c, w2v_c, us, is_, w1u, b1u, w1v, b1v,
                w2u_s, w2v_s):
    return pl.pallas_call(
        _embed_body,
        out_shape=[
            jax.ShapeDtypeStruct((N_USERS, ENC), jnp.float32),
            jax.ShapeDtypeStruct((N_ITEMS, ENC), jnp.float32),
        ],
    )(uh, ih, w2u_c, w2v_c, us, is_, w1u, b1u, w1v, b1v, w2u_s, w2v_s)


def _basis_body(ue_ref, wdec_ref, ie_ref, out_ref):
    for b in range(NBAS):
        uw = jnp.dot(ue_ref[...], wdec_ref[b],
                     preferred_element_type=jnp.float32, precision=_HI)
        out_ref[b] = jax.lax.dot_general(
            uw, ie_ref[...], (((1,), (1,)), ((), ())),
            preferred_element_type=jnp.float32, precision=_HI)


def _basis_table(ue, wdec, ie, bu=256):
    return pl.pallas_call(
        _basis_body,
        grid=(N_USERS // bu,),
        in_specs=[
            pl.BlockSpec((bu, ENC), lambda j: (j, 0)),
            pl.BlockSpec((NBAS, ENC, ENC), lambda j: (0, 0, 0)),
            pl.BlockSpec((N_ITEMS, ENC), lambda j: (0, 0)),
        ],
        out_specs=pl.BlockSpec((NBAS, bu, N_ITEMS), lambda j: (0, j, 0)),
        out_shape=jax.ShapeDtypeStruct((NBAS, N_USERS, N_ITEMS), jnp.float32),
    )(ue, wdec, ie)


def kernel(user_supports, item_supports, user_inputs, item_inputs,
           user_side_inputs, item_side_inputs, user_edge_idx, item_edge_idx,
           labels, W_gcn, W1_u, b1_u, W1_v, b1_v, W2_u, W2_v, W_dec, W_cls):
    tok = jnp.asarray(_TOKEN_NODES)
    noi = jnp.asarray(_NOISE_NODES)
    cho = jnp.asarray(_NOISE_CHOSEN)
    uidx = user_edge_idx.astype(jnp.int32)
    midx = item_edge_idx.astype(jnp.int32)
    labels = labels.astype(jnp.int32)

    pre_uidx = uidx.at[tok].set(-1).at[noi].set(uidx[cho])
    pre_midx = midx.at[tok].set(-1).at[noi].set(midx[cho])

    rt = labels[tok]; ut = uidx[tok]; mt = midx[tok]
    rn = labels[noi]; un = uidx[noi]; mn = midx[noi]
    rN = labels[cho]; uN = uidx[cho]; mN = midx[cho]

    usup_f = user_supports.reshape(-1)
    isup_f = item_supports.reshape(-1)

    n_tok = tok.shape[0]

    def direction(flat, tok_ids, noi_ids, src_ids):
        ids = jnp.concatenate([tok_ids, noi_ids])
        seq = jnp.arange(ids.shape[0], dtype=jnp.int32)
        order = jnp.argsort(ids, stable=True)
        ids_s = ids[order]; seq_s = seq[order]
        pos = jnp.searchsorted(ids_s, src_ids)
        pos = jnp.clip(pos, 0, ids_s.shape[0] - 1)
        src_zeroed = (ids_s[pos] == src_ids) & (seq_s[pos] < n_tok)
        vals = jnp.where(src_zeroed, 0.0, flat[src_ids])
        upd = jnp.concatenate([jnp.zeros_like(tok_ids, jnp.float32), vals])
        upd_s = upd[order]
        is_last = jnp.concatenate([ids_s[1:] != ids_s[:-1],
                                   jnp.ones((1,), bool)])
        delta = jnp.where(is_last, upd_s - flat[ids_s], 0.0)
        return ids_s, delta

    idu_t = (rt * N_USERS + ut) * N_ITEMS + mt
    idu_n = (rn * N_USERS + un) * N_ITEMS + mn
    idu_s = (rN * N_USERS + uN) * N_ITEMS + mN
    ids_u, delta_u = direction(usup_f, idu_t, idu_n, idu_s)

    idm_t = (rt * N_ITEMS + mt) * N_USERS + ut
    idm_n = (rn * N_ITEMS + mn) * N_USERS + un
    idm_s = (rN * N_ITEMS + mN) * N_USERS + uN
    ids_m, delta_m = direction(isup_f, idm_t, idm_n, idm_s)

    wg_pad = jnp.pad(W_gcn, ((0, 0), (0, 0), (0, CHP - CH)))
    TU, TV = _input_transforms(user_inputs, item_inputs, wg_pad)
    uh = _spmm(user_supports, TV, N_USERS, N_ITEMS, 256)
    ih = _spmm(item_supports, TU, N_ITEMS, N_USERS, 256)

    r_u = ids_u // (N_USERS * N_ITEMS)
    rem = ids_u % (N_USERS * N_ITEMS)
    u_u = rem // N_ITEMS
    m_u = rem % N_ITEMS
    uh = uh.at[r_u, u_u].add(delta_u[:, None] * TV[r_u, m_u])

    r_m = ids_m // (N_ITEMS * N_USERS)
    rem = ids_m % (N_ITEMS * N_USERS)
    m_m = rem // N_USERS
    u_m = rem % N_USERS
    ih = ih.at[r_m, m_m].add(delta_m[:, None] * TU[r_m, u_m])

    w2u_c = jnp.pad(W2_u[:NSUP * CH].reshape(NSUP, CH, ENC),
                    ((0, 0), (0, CHP - CH), (0, 0)))
    w2v_c = jnp.pad(W2_v[:NSUP * CH].reshape(NSUP, CH, ENC),
                    ((0, 0), (0, CHP - CH), (0, 0)))
    ue, ie = _embeddings(uh, ih, w2u_c, w2v_c,
                         user_side_inputs, item_side_inputs,
                         W1_u, b1_u.reshape(1, SIDE_H),
                         W1_v, b1_v.reshape(1, SIDE_H),
                         W2_u[NSUP * CH:], W2_v[NSUP * CH:])

    btab = _basis_table(ue, W_dec, ie)

    uw = pre_uidx % N_USERS
    mw = pre_midx % N_ITEMS
    flat = uw * N_ITEMS + mw
    basis = btab.reshape(NBAS, -1)[:, flat]
    return basis.T @ W_cls


# speedup vs baseline: 1.4393x; 1.4393x over previous
import functools

import numpy as np
import jax
import jax.numpy as jnp
from jax import lax
from jax.experimental import pallas as pl
from jax.experimental.pallas import tpu as pltpu
from jax.experimental.pallas import tpu_sc as plsc

N_USERS = 2048
N_ITEMS = 1024
NR = 160000
NSUP = 5
NCLS = 5
NBAS = 3
INPUT_DIM = 512
CH = 100
CHP = 128
SIDE_H = 64
ENC = 128

NP_TAB = N_USERS * N_ITEMS
NW = 32
EP = 163840
EPT = EP // NW
NCHK = EPT // 128

_HI = jax.lax.Precision.HIGHEST


def _mask_constants():
    mkey = jax.random.key(12345)
    k1, k2, k3 = jax.random.split(mkey, 3)
    perm = jax.random.permutation(k1, NR)
    num_mask = 48000
    num_noise = 4800
    num_token = 43200
    mask_nodes = perm[:num_mask]
    perm_mask = jax.random.permutation(k2, num_mask)
    token_nodes = mask_nodes[perm_mask[:num_token]]
    noise_nodes = mask_nodes[perm_mask[num_mask - num_noise:]]
    noise_chosen = jax.random.permutation(k3, NR)[:num_noise]
    return (np.asarray(token_nodes), np.asarray(noise_nodes),
            np.asarray(noise_chosen))


_TOKEN_NODES, _NOISE_NODES, _NOISE_CHOSEN = _mask_constants()


def _k1_body(ui_ref, ii_ref, wg_ref, tu_ref, tv_ref):
    w = wg_ref[0]
    tu_ref[0] = jnp.dot(ui_ref[...], w, preferred_element_type=jnp.float32,
                        precision=_HI)
    tv_ref[0] = jnp.dot(ii_ref[...], w, preferred_element_type=jnp.float32,
                        precision=_HI)


def _input_transforms(user_inputs, item_inputs, wg_pad):
    return pl.pallas_call(
        _k1_body,
        grid=(NSUP,),
        in_specs=[
            pl.BlockSpec((N_USERS, INPUT_DIM), lambda i: (0, 0)),
            pl.BlockSpec((N_ITEMS, INPUT_DIM), lambda i: (0, 0)),
            pl.BlockSpec((1, INPUT_DIM, CHP), lambda i: (i, 0, 0)),
        ],
        out_specs=[
            pl.BlockSpec((1, N_USERS, CHP), lambda i: (i, 0, 0)),
            pl.BlockSpec((1, N_ITEMS, CHP), lambda i: (i, 0, 0)),
        ],
        out_shape=[
            jax.ShapeDtypeStruct((NSUP, N_USERS, CHP), jnp.float32),
            jax.ShapeDtypeStruct((NSUP, N_ITEMS, CHP), jnp.float32),
        ],
    )(user_inputs, item_inputs, wg_pad)


def _spmm_body(sup_ref, t_ref, out_ref):
    out_ref[0] = jnp.dot(sup_ref[0], t_ref[0],
                         preferred_element_type=jnp.float32, precision=_HI)


def _spmm(sup, t, n_rows, n_cols, bu):
    return pl.pallas_call(
        _spmm_body,
        grid=(NSUP, n_rows // bu),
        in_specs=[
            pl.BlockSpec((1, bu, n_cols), lambda i, j: (i, j, 0)),
            pl.BlockSpec((1, n_cols, CHP), lambda i, j: (i, 0, 0)),
        ],
        out_specs=pl.BlockSpec((1, bu, CHP), lambda i, j: (i, j, 0)),
        out_shape=jax.ShapeDtypeStruct((NSUP, n_rows, CHP), jnp.float32),
    )(sup, t)


def _embed_body(uh_ref, ih_ref, w2u_ref, w2v_ref, us_ref, is_ref,
                w1u_ref, b1u_ref, w1v_ref, b1v_ref, w2us_ref, w2vs_ref,
                ue_ref, ie_ref):
    ue = jnp.zeros((N_USERS, ENC), jnp.float32)
    ie = jnp.zeros((N_ITEMS, ENC), jnp.float32)
    for i in range(NSUP):
        ue += jnp.dot(jnp.maximum(uh_ref[i], 0.0), w2u_ref[i],
                      preferred_element_type=jnp.float32, precision=_HI)
        ie += jnp.dot(jnp.maximum(ih_ref[i], 0.0), w2v_ref[i],
                      preferred_element_type=jnp.float32, precision=_HI)
    us = jnp.maximum(jnp.dot(us_ref[...], w1u_ref[...],
                             preferred_element_type=jnp.float32,
                             precision=_HI) + b1u_ref[...], 0.0)
    vs = jnp.maximum(jnp.dot(is_ref[...], w1v_ref[...],
                             preferred_element_type=jnp.float32,
                             precision=_HI) + b1v_ref[...], 0.0)
    ue += jnp.dot(us, w2us_ref[...], preferred_element_type=jnp.float32,
                  precision=_HI)
    ie += jnp.dot(vs, w2vs_ref[...], preferred_element_type=jnp.float32,
                  precision=_HI)
    ue_ref[...] = ue
    ie_ref[...] = ie


def _embeddings(uh, ih, w2u_c, w2v_c, us, is_, w1u, b1u, w1v, b1v,
                w2u_s, w2v_s):
    return pl.pallas_call(
        _embed_body,
        out_shape=[
            jax.ShapeDtypeStruct((N_USERS, ENC), jnp.float32),
            jax.ShapeDtypeStruct((N_ITEMS, ENC), jnp.float32),
        ],
    )(uh, ih, w2u_c, w2v_c, us, is_, w1u, b1u, w1v, b1v, w2u_s, w2v_s)


def _basis_body(ue_ref, wdec_ref, ie_ref, out_ref):
    for b in range(NBAS):
        uw = jnp.dot(ue_ref[...], wdec_ref[b],
                     preferred_element_type=jnp.float32, precision=_HI)
        out_ref[b] = jax.lax.dot_general(
            uw, ie_ref[...], (((1,), (1,)), ((), ())),
            preferred_element_type=jnp.float32, precision=_HI)


def _basis_table(ue, wdec, ie, bu=256):
    return pl.pallas_call(
        _basis_body,
        grid=(N_USERS // bu,),
        in_specs=[
            pl.BlockSpec((bu, ENC), lambda j: (j, 0)),
            pl.BlockSpec((NBAS, ENC, ENC), lambda j: (0, 0, 0)),
            pl.BlockSpec((N_ITEMS, ENC), lambda j: (0, 0)),
        ],
        out_specs=pl.BlockSpec((NBAS, bu, N_ITEMS), lambda j: (0, j, 0)),
        out_shape=jax.ShapeDtypeStruct((NBAS, N_USERS, N_ITEMS), jnp.float32),
    )(ue, wdec, ie)


def _dec_body(btab, idxs, wspl, out, idx_v, w_v, g0, g1, g2,
              o0, o1, o2, o3, o4, sem):
    c = lax.axis_index("c")
    s = lax.axis_index("s")
    wid = s * 2 + c
    pltpu.sync_copy(idxs.at[wid], idx_v)
    pltpu.sync_copy(wspl, w_v)
    gbufs = (g0, g1, g2)
    copies = []
    for b in range(NBAS):
        for j in range(NCHK):
            copies.append(pltpu.async_copy(
                btab.at[idx_v.at[b * NCHK + j]],
                gbufs[b].at[pl.ds(j * 128, 128)], sem))
    for cp in copies:
        cp.wait()

    w = [w_v[pl.ds(i * 16, 16)] for i in range(NBAS * NCLS)]
    obufs = (o0, o1, o2, o3, o4)

    def step(i, _):
        k = i * 16
        vb = [gbufs[b][pl.ds(k, 16)] for b in range(NBAS)]
        for cl in range(NCLS):
            acc = w[cl] * vb[0]
            acc = acc + w[NCLS + cl] * vb[1]
            acc = acc + w[2 * NCLS + cl] * vb[2]
            obufs[cl][pl.ds(k, 16)] = acc
        return 0

    lax.fori_loop(0, EPT // 16, step, 0)
    for cl in range(NCLS):
        pltpu.sync_copy(obufs[cl], out.at[pl.ds(cl * EP + wid * EPT, EPT)])


def _decoder_sc(btab_flat, idx_tiles, wspl):
    mesh = plsc.VectorSubcoreMesh(core_axis_name="c", subcore_axis_name="s")
    f = pl.kernel(
        _dec_body, mesh=mesh,
        out_type=jax.ShapeDtypeStruct((NCLS * EP,), jnp.float32),
        scratch_types=[
            pltpu.VMEM((NBAS * NCHK, 128), jnp.int32),
            pltpu.VMEM((256,), jnp.float32),
            pltpu.VMEM((EPT,), jnp.float32),
            pltpu.VMEM((EPT,), jnp.float32),
            pltpu.VMEM((EPT,), jnp.float32),
            pltpu.VMEM((EPT,), jnp.float32),
            pltpu.VMEM((EPT,), jnp.float32),
            pltpu.VMEM((EPT,), jnp.float32),
            pltpu.VMEM((EPT,), jnp.float32),
            pltpu.VMEM((EPT,), jnp.float32),
            pltpu.SemaphoreType.DMA,
        ],
    )
    return f(btab_flat, idx_tiles, wspl)


def kernel(user_supports, item_supports, user_inputs, item_inputs,
           user_side_inputs, item_side_inputs, user_edge_idx, item_edge_idx,
           labels, W_gcn, W1_u, b1_u, W1_v, b1_v, W2_u, W2_v, W_dec, W_cls):
    tok = jnp.asarray(_TOKEN_NODES)
    noi = jnp.asarray(_NOISE_NODES)
    cho = jnp.asarray(_NOISE_CHOSEN)
    uidx = user_edge_idx.astype(jnp.int32)
    midx = item_edge_idx.astype(jnp.int32)
    labels = labels.astype(jnp.int32)

    pre_uidx = uidx.at[tok].set(-1).at[noi].set(uidx[cho])
    pre_midx = midx.at[tok].set(-1).at[noi].set(midx[cho])

    rt = labels[tok]; ut = uidx[tok]; mt = midx[tok]
    rn = labels[noi]; un = uidx[noi]; mn = midx[noi]
    rN = labels[cho]; uN = uidx[cho]; mN = midx[cho]

    usup_f = user_supports.reshape(-1)
    isup_f = item_supports.reshape(-1)

    n_tok = tok.shape[0]

    def direction(flat, tok_ids, noi_ids, src_ids):
        ids = jnp.concatenate([tok_ids, noi_ids])
        seq = jnp.arange(ids.shape[0], dtype=jnp.int32)
        order = jnp.argsort(ids, stable=True)
        ids_s = ids[order]; seq_s = seq[order]
        pos = jnp.searchsorted(ids_s, src_ids)
        pos = jnp.clip(pos, 0, ids_s.shape[0] - 1)
        src_zeroed = (ids_s[pos] == src_ids) & (seq_s[pos] < n_tok)
        vals = jnp.where(src_zeroed, 0.0, flat[src_ids])
        upd = jnp.concatenate([jnp.zeros_like(tok_ids, jnp.float32), vals])
        upd_s = upd[order]
        is_last = jnp.concatenate([ids_s[1:] != ids_s[:-1],
                                   jnp.ones((1,), bool)])
        delta = jnp.where(is_last, upd_s - flat[ids_s], 0.0)
        return ids_s, delta

    idu_t = (rt * N_USERS + ut) * N_ITEMS + mt
    idu_n = (rn * N_USERS + un) * N_ITEMS + mn
    idu_s = (rN * N_USERS + uN) * N_ITEMS + mN
    ids_u, delta_u = direction(usup_f, idu_t, idu_n, idu_s)

    idm_t = (rt * N_ITEMS + mt) * N_USERS + ut
    idm_n = (rn * N_ITEMS + mn) * N_USERS + un
    idm_s = (rN * N_ITEMS + mN) * N_USERS + uN
    ids_m, delta_m = direction(isup_f, idm_t, idm_n, idm_s)

    wg_pad = jnp.pad(W_gcn, ((0, 0), (0, 0), (0, CHP - CH)))
    TU, TV = _input_transforms(user_inputs, item_inputs, wg_pad)
    uh = _spmm(user_supports, TV, N_USERS, N_ITEMS, 256)
    ih = _spmm(item_supports, TU, N_ITEMS, N_USERS, 256)

    r_u = ids_u // (N_USERS * N_ITEMS)
    rem = ids_u % (N_USERS * N_ITEMS)
    u_u = rem // N_ITEMS
    m_u = rem % N_ITEMS
    uh = uh.at[r_u, u_u].add(delta_u[:, None] * TV[r_u, m_u])

    r_m = ids_m // (N_ITEMS * N_USERS)
    rem = ids_m % (N_ITEMS * N_USERS)
    m_m = rem // N_USERS
    u_m = rem % N_USERS
    ih = ih.at[r_m, m_m].add(delta_m[:, None] * TU[r_m, u_m])

    w2u_c = jnp.pad(W2_u[:NSUP * CH].reshape(NSUP, CH, ENC),
                    ((0, 0), (0, CHP - CH), (0, 0)))
    w2v_c = jnp.pad(W2_v[:NSUP * CH].reshape(NSUP, CH, ENC),
                    ((0, 0), (0, CHP - CH), (0, 0)))
    ue, ie = _embeddings(uh, ih, w2u_c, w2v_c,
                         user_side_inputs, item_side_inputs,
                         W1_u, b1_u.reshape(1, SIDE_H),
                         W1_v, b1_v.reshape(1, SIDE_H),
                         W2_u[NSUP * CH:], W2_v[NSUP * CH:])

    btab = _basis_table(ue, W_dec, ie)

    uw = pre_uidx % N_USERS
    mw = pre_midx % N_ITEMS
    flat = uw * N_ITEMS + mw
    flat3 = flat[None, :] + (jnp.arange(NBAS, dtype=jnp.int32)
                             * NP_TAB)[:, None]
    flat3 = jnp.pad(flat3, ((0, 0), (0, EP - NR)))
    idx_tiles = (flat3.reshape(NBAS, NW, NCHK, 128)
                 .transpose(1, 0, 2, 3)
                 .reshape(NW, NBAS * NCHK, 128))
    wspl = jnp.pad(jnp.repeat(W_cls.reshape(-1), 16), (0, 16))
    out = _decoder_sc(btab.reshape(-1), idx_tiles, wspl)
    return out.reshape(NCLS, EP)[:, :NR].T
